# baseline (speedup 1.0000x reference)
import jax
import jax.numpy as jnp
from jax import lax
from jax.experimental import pallas as pl
from jax.experimental.pallas import tpu as pltpu
from jax.experimental.pallas import tpu_sc as plsc

VOCAB = 1000000
EMBED = 32
BATCH = 16384
HIST = 50

NC, NS = 2, 16
NW = NC * NS
ROWS_PER_W = BATCH // NW
IDX_PER_W = ROWS_PER_W * HIST
NBUF = 2
TE = EMBED // 8
TB = ROWS_PER_W // 128
TBS = 513
UNROLL = 16


def _gather_body(idx_hbm, table_hbm, out_hbm, idx_raw, idx_t, gbuf, tbuf,
                 sem_g, sem_o):
    wid = lax.axis_index("s") * NC + lax.axis_index("c")
    iota = lax.iota(jnp.int32, 16)

    pltpu.sync_copy(idx_hbm.at[pl.ds(wid * IDX_PER_W, IDX_PER_W)], idx_raw)

    @plsc.parallel_loop(0, IDX_PER_W // 16, 1, unroll=8)
    def tr_idx(p):
        pos = p * 16 + iota
        vals = idx_raw[pl.ds(p * 16, 16)]
        b = pos // HIST
        h = pos - b * HIST
        plsc.store_scatter(idx_t, [h * ROWS_PER_W + b], vals)

    def start_gather(h, s):
        pltpu.async_copy(
            table_hbm.at[idx_t.at[pl.ds(h * ROWS_PER_W, ROWS_PER_W)]],
            gbuf.at[s],
            sem_g.at[s],
        )

    def wait_gather(s):
        pltpu.make_async_copy(
            table_hbm.at[pl.ds(0, ROWS_PER_W)], gbuf.at[s], sem_g.at[s]
        ).wait()

    def start_writes(h, s):
        for te in range(TE):
            for tbi in range(TB):
                pltpu.async_copy(
                    tbuf.at[s].at[pl.ds(te * 8, 8), pl.ds(tbi * 128, 128)],
                    out_hbm.at[h].at[te].at[wid * TB + tbi],
                    sem_o.at[s],
                )

    def wait_writes(s):
        for _ in range(TE * TB):
            pltpu.make_async_copy(
                tbuf.at[s].at[pl.ds(0, 8), pl.ds(0, 128)],
                out_hbm.at[0].at[0].at[0],
                sem_o.at[s],
            ).wait()

    for s in range(NBUF):
        start_gather(s, s)

    def step(h, _):
        s = lax.rem(h, NBUF)
        wait_gather(s)

        @pl.when(h >= NBUF)
        def _():
            wait_writes(s)

        @plsc.parallel_loop(0, ROWS_PER_W, 1, unroll=UNROLL)
        def tr_rows(b):
            bvec = lax.broadcast(b, (16,)).astype(jnp.int32)
            v0 = plsc.load_gather(gbuf.at[s], [bvec, iota])
            v1 = plsc.load_gather(gbuf.at[s], [bvec, iota + 16])
            plsc.store_scatter(tbuf.at[s], [iota, bvec], v0)
            plsc.store_scatter(tbuf.at[s], [iota + 16, bvec], v1)

        @pl.when(h + NBUF < HIST)
        def _():
            start_gather(h + NBUF, s)

        start_writes(h, s)
        return 0

    lax.fori_loop(0, HIST, step, 0)

    for s in range(NBUF):
        wait_writes(s)


@jax.jit
def kernel(x, W):
    idx = x.reshape(-1).astype(jnp.int32)
    mesh = plsc.VectorSubcoreMesh(core_axis_name="c", subcore_axis_name="s")
    P = pl.kernel(
        _gather_body,
        out_type=jax.ShapeDtypeStruct((HIST, TE, BATCH // 128, 8, 128),
                                      jnp.float32),
        mesh=mesh,
        scratch_types=[
            pltpu.VMEM((IDX_PER_W,), jnp.int32),
            pltpu.VMEM((IDX_PER_W,), jnp.int32),
            pltpu.VMEM((NBUF, ROWS_PER_W, EMBED), jnp.float32),
            pltpu.VMEM((NBUF, EMBED, TBS), jnp.float32),
            pltpu.SemaphoreType.DMA((NBUF,)),
            pltpu.SemaphoreType.DMA((NBUF,)),
        ],
        compiler_params=pltpu.CompilerParams(use_tc_tiling_on_sc=False, needs_layout_passes=False),
    )(idx, W)
    return P.transpose(2, 4, 0, 1, 3).reshape(BATCH, HIST, EMBED)


# speedup vs baseline: 1.0426x; 1.0426x over previous
import jax
import jax.numpy as jnp
from jax import lax
from jax.experimental import pallas as pl
from jax.experimental.pallas import tpu as pltpu
from jax.experimental.pallas import tpu_sc as plsc

VOCAB = 1000000
EMBED = 32
BATCH = 16384
HIST = 50

NC, NS = 2, 16
NW = NC * NS
ROWS_PER_W = BATCH // NW
IDX_PER_W = ROWS_PER_W * HIST
NBUF = 2
TE = EMBED // 8
TB = ROWS_PER_W // 128
TBS = 513
UNROLL = 8


def _gather_body(idx_hbm, table_hbm, out_hbm, idx_raw, idx_t, gbuf, tbuf,
                 sem_g, sem_o):
    wid = lax.axis_index("s") * NC + lax.axis_index("c")
    iota = lax.iota(jnp.int32, 16)

    pltpu.sync_copy(idx_hbm.at[pl.ds(wid * IDX_PER_W, IDX_PER_W)], idx_raw)

    @plsc.parallel_loop(0, IDX_PER_W // 16, 1, unroll=4)
    def tr_idx(p):
        pos = p * 16 + iota
        vals = idx_raw[pl.ds(p * 16, 16)]
        b = pos // HIST
        h = pos - b * HIST
        plsc.store_scatter(idx_t, [h * ROWS_PER_W + b], vals)

    def start_gather(h, s):
        pltpu.async_copy(
            table_hbm.at[idx_t.at[pl.ds(h * ROWS_PER_W, ROWS_PER_W)]],
            gbuf.at[s],
            sem_g.at[s],
        )

    def wait_gather(s):
        pltpu.make_async_copy(
            table_hbm.at[pl.ds(0, ROWS_PER_W)], gbuf.at[s], sem_g.at[s]
        ).wait()

    def start_writes(h, s):
        for te in range(TE):
            for tbi in range(TB):
                pltpu.async_copy(
                    tbuf.at[s].at[pl.ds(te * 8, 8), pl.ds(tbi * 128, 128)],
                    out_hbm.at[h].at[te].at[wid * TB + tbi],
                    sem_o.at[s],
                )

    def wait_writes(s):
        for _ in range(TE * TB):
            pltpu.make_async_copy(
                tbuf.at[s].at[pl.ds(0, 8), pl.ds(0, 128)],
                out_hbm.at[0].at[0].at[0],
                sem_o.at[s],
            ).wait()

    for s in range(NBUF):
        start_gather(s, s)

    def step(h, _):
        s = lax.rem(h, NBUF)
        wait_gather(s)

        @pl.when(h >= NBUF)
        def _():
            wait_writes(s)

        @plsc.parallel_loop(0, ROWS_PER_W, 1, unroll=UNROLL)
        def tr_rows(b):
            bvec = lax.broadcast(b, (16,)).astype(jnp.int32)
            v0 = plsc.load_gather(gbuf.at[s], [bvec, iota])
            v1 = plsc.load_gather(gbuf.at[s], [bvec, iota + 16])
            plsc.store_scatter(tbuf.at[s], [iota, bvec], v0)
            plsc.store_scatter(tbuf.at[s], [iota + 16, bvec], v1)

        @pl.when(h + NBUF < HIST)
        def _():
            start_gather(h + NBUF, s)

        start_writes(h, s)
        return 0

    lax.fori_loop(0, HIST, step, 0)

    for s in range(NBUF):
        wait_writes(s)


@jax.jit
def kernel(x, W):
    idx = x.reshape(-1).astype(jnp.int32)
    mesh = plsc.VectorSubcoreMesh(core_axis_name="c", subcore_axis_name="s")
    P = pl.kernel(
        _gather_body,
        out_type=jax.ShapeDtypeStruct((HIST, TE, BATCH // 128, 8, 128),
                                      jnp.float32),
        mesh=mesh,
        scratch_types=[
            pltpu.VMEM((IDX_PER_W,), jnp.int32),
            pltpu.VMEM((IDX_PER_W,), jnp.int32),
            pltpu.VMEM((NBUF, ROWS_PER_W, EMBED), jnp.float32),
            pltpu.VMEM((NBUF, EMBED, TBS), jnp.float32),
            pltpu.SemaphoreType.DMA((NBUF,)),
            pltpu.SemaphoreType.DMA((NBUF,)),
        ],
        compiler_params=pltpu.CompilerParams(use_tc_tiling_on_sc=False, needs_layout_passes=False),
    )(idx, W)
    return P.transpose(2, 4, 0, 1, 3).reshape(BATCH, HIST, EMBED)
